# baseline (speedup 1.0000x reference)
import functools

import jax
import jax.numpy as jnp
from jax import lax
from jax.experimental import pallas as pl
from jax.experimental.pallas import tpu as pltpu
from jax.experimental.pallas import tpu_sc as plsc

N = 10000
C = 256
H = 8
D = 32
E = 160000
AP = 16
LW = 128

NW = 32
ROWS = 320
OUTP = NW * ROWS
BLK = 48
PE = E + 384
VL = 16
XC = 384


def _make_edge_body(w_off):
    def body(pk_h, st_h, ct_h, adfl_h, hx_h, out_h,
             acc, s_flat, adst_loc, idx_rc0, idx_adj0, idx_lkp0,
             idx_rc1, idx_adj1, idx_lkp1, hbuf0, hbuf1, mbuf, sem0, sem1):
        c = lax.axis_index("c")
        s = lax.axis_index("s")
        wid = c * 16 + s
        base = wid * ROWS

        pltpu.sync_copy(st_h.at[pl.ds(wid * 8, VL)], mbuf)
        st = pl.multiple_of(mbuf[...][0], 8)
        pltpu.sync_copy(ct_h.at[pl.ds(wid * 8, VL)], mbuf)
        cnt = mbuf[...][0]
        nb = (cnt + (BLK - 1)) // BLK

        pltpu.sync_copy(adfl_h.at[pl.ds(base * AP, ROWS * AP)], adst_loc)

        def za(i, carry):
            acc[pl.ds(i * VL, VL)] = jnp.zeros((VL,), jnp.float32)
            return carry

        lax.fori_loop(0, (ROWS + 1) * C // VL, za, 0)

        def zs(i, carry):
            s_flat[pl.ds(i * VL, VL)] = jnp.zeros((VL,), jnp.float32)
            return carry

        lax.fori_loop(0, (ROWS + 1) * AP // VL, zs, 0)

        def load_issue(b, idx_rc, idx_adj, idx_lkp, hbuf, sem):
            off = st + b * BLK
            pltpu.sync_copy(pk_h.at[pl.ds(off, BLK)], idx_rc)
            for i in range(BLK // VL):
                pv = idx_rc[pl.ds(i * VL, VL)]
                cv = pv >> 16
                rv = pv & 0xFFFF
                adj = cv - base
                ok = (adj >= 0) & (adj < ROWS)
                idx_adj[pl.ds(i * VL, VL)] = jnp.where(ok, adj, ROWS)
                idx_lkp[pl.ds(i * VL, VL)] = jnp.where(ok, adj, 0)
                idx_rc[pl.ds(i * VL, VL)] = rv
            pltpu.async_copy(hx_h.at[idx_rc], hbuf, sem)

        def compute(idx_adj, idx_lkp, hbuf):
            def grp(g, cg):
                iav = idx_adj[pl.ds(g * VL, VL)]
                ilv = idx_lkp[pl.ds(g * VL, VL)]
                for j in range(VL):
                    e = g * VL + j
                    av = (hbuf[e, pl.ds(w_off, VL)]
                          + adst_loc[pl.ds(ilv[j] * AP, VL)])
                    av = jnp.where(av >= 0.0, av, 0.2 * av)
                    wv = jnp.exp(av)
                    sofs = iav[j] * AP
                    s_flat[pl.ds(sofs, VL)] = s_flat[pl.ds(sofs, VL)] + wv
                    aofs = iav[j] * C
                    for hd in range(H):
                        w = wv[hd]
                        for q in range(2):
                            so = hd * D + q * VL
                            dsl = pl.ds(aofs + so, VL)
                            acc[dsl] = acc[dsl] + hbuf[e, pl.ds(so, VL)] * w
                return cg

            lax.fori_loop(0, BLK // VL, grp, 0)

        def wait(hbuf, sem):
            pltpu.make_async_copy(hx_h.at[idx_rc0], hbuf, sem).wait()

        @pl.when(nb > 0)
        def _():
            load_issue(0, idx_rc0, idx_adj0, idx_lkp0, hbuf0, sem0)

        @pl.when(nb > 1)
        def _():
            load_issue(1, idx_rc1, idx_adj1, idx_lkp1, hbuf1, sem1)

        def pair(g, carry):
            b0 = 2 * g
            b1 = b0 + 1

            @pl.when(b0 < nb)
            def _():
                wait(hbuf0, sem0)
                compute(idx_adj0, idx_lkp0, hbuf0)

                @pl.when(b0 + 2 < nb)
                def _():
                    load_issue(b0 + 2, idx_rc0, idx_adj0, idx_lkp0, hbuf0, sem0)

            @pl.when(b1 < nb)
            def _():
                wait(hbuf1, sem1)
                compute(idx_adj1, idx_lkp1, hbuf1)

                @pl.when(b1 + 2 < nb)
                def _():
                    load_issue(b1 + 2, idx_rc1, idx_adj1, idx_lkp1, hbuf1, sem1)

            return carry

        lax.fori_loop(0, (nb + 1) // 2, pair, 0)

        def nrm(r, cn):
            sv = s_flat[pl.ds(r * VL, VL)]
            iv = 1.0 / (sv + 1e-16)
            for hd in range(H):
                w = iv[hd]
                for q in range(2):
                    so = r * C + hd * D + q * VL
                    v = acc[pl.ds(so, VL)] * w
                    acc[pl.ds(so, VL)] = jnp.maximum(v, 0.0)
            return cn

        lax.fori_loop(0, ROWS, nrm, 0)
        pltpu.sync_copy(acc.at[pl.ds(0, ROWS * C)],
                        out_h.at[pl.ds(base * C, ROWS * C)])

    return body


def _make_edge_pass(w_off):
    return functools.partial(
        pl.kernel,
        mesh=plsc.VectorSubcoreMesh(core_axis_name="c", subcore_axis_name="s"),
        out_type=jax.ShapeDtypeStruct((OUTP * C,), jnp.float32),
        scratch_types=[
            pltpu.VMEM(((ROWS + 1) * C,), jnp.float32),
            pltpu.VMEM(((ROWS + 1) * AP,), jnp.float32),
            pltpu.VMEM((ROWS * AP,), jnp.float32),
            pltpu.VMEM((BLK,), jnp.int32),
            pltpu.VMEM((BLK,), jnp.int32),
            pltpu.VMEM((BLK,), jnp.int32),
            pltpu.VMEM((BLK,), jnp.int32),
            pltpu.VMEM((BLK,), jnp.int32),
            pltpu.VMEM((BLK,), jnp.int32),
            pltpu.VMEM((BLK, XC), jnp.float32),
            pltpu.VMEM((BLK, XC), jnp.float32),
            pltpu.VMEM((VL,), jnp.int32),
            pltpu.SemaphoreType.DMA,
            pltpu.SemaphoreType.DMA,
        ],
    )(_make_edge_body(w_off))


_edge_pass_s0 = _make_edge_pass(C)
_edge_pass_s1 = _make_edge_pass(C + AP)


def _edge_conv(bkt, edge_pass, adfl, hx):
    pk, st8, ct8 = bkt
    out = edge_pass(pk, st8, ct8, adfl, hx)
    return out.reshape(OUTP, C)[:N]


def _bucketize(ei):
    row, col = ei[0], ei[1]
    key = col // ROWS
    order = jnp.argsort(key)
    row_s = jnp.take(row, order)
    col_s = jnp.take(col, order)
    key_s = jnp.take(key, order)
    cnt = jnp.bincount(key, length=NW).astype(jnp.int32)
    pcnt = ((cnt + 7) // 8) * 8
    st = (jnp.cumsum(pcnt) - pcnt).astype(jnp.int32)
    cst = (jnp.cumsum(cnt) - cnt).astype(jnp.int32)
    rank = jnp.arange(E, dtype=jnp.int32) - jnp.take(cst, key_s)
    dest = jnp.take(st, key_s) + rank
    pk_s = (col_s << 16) | row_s
    pk = jnp.full((PE,), -65536, jnp.int32).at[dest].set(pk_s)
    st8 = jnp.pad(jnp.repeat(st, 8), (0, 16))
    ct8 = jnp.pad(jnp.repeat(cnt, 8), (0, 16))
    return pk, st8, ct8


BN = 400


def _proj_body(x_ref, w_ref, b_ref, as0_ref, as1_ref, ad0_ref, ad1_ref,
               hx_ref, od0_ref, od1_ref):
    h = jnp.dot(x_ref[...], w_ref[...],
                preferred_element_type=jnp.float32) + b_ref[...]
    a_s0 = jnp.dot(h, as0_ref[...], preferred_element_type=jnp.float32)
    a_s1 = jnp.dot(h, as1_ref[...], preferred_element_type=jnp.float32)
    zz = jnp.zeros((BN, XC - C - 2 * AP), jnp.float32)
    hx_ref[...] = jnp.concatenate([h, a_s0, a_s1, zz], axis=1)
    od0_ref[...] = jnp.dot(h, ad0_ref[...], preferred_element_type=jnp.float32)
    od1_ref[...] = jnp.dot(h, ad1_ref[...], preferred_element_type=jnp.float32)


_proj = pl.pallas_call(
    _proj_body,
    grid=(N // BN,),
    in_specs=[pl.BlockSpec((BN, C), lambda i: (i, 0)),
              pl.BlockSpec((C, C), lambda i: (0, 0)),
              pl.BlockSpec((1, C), lambda i: (0, 0))]
             + [pl.BlockSpec((C, AP), lambda i: (0, 0))] * 4,
    out_specs=[pl.BlockSpec((BN, XC), lambda i: (i, 0)),
               pl.BlockSpec((BN, AP), lambda i: (i, 0)),
               pl.BlockSpec((BN, AP), lambda i: (i, 0))],
    out_shape=[jax.ShapeDtypeStruct((N, XC), jnp.float32),
               jax.ShapeDtypeStruct((N, AP), jnp.float32),
               jax.ShapeDtypeStruct((N, AP), jnp.float32)],
)


def _score_body(o0_ref, o1_ref, wk_ref, bk_ref, q_ref, out_ref):
    i = pl.program_id(0)
    t0 = jnp.tanh(jnp.dot(o0_ref[...], wk_ref[...],
                          preferred_element_type=jnp.float32) + bk_ref[...])
    t1 = jnp.tanh(jnp.dot(o1_ref[...], wk_ref[...],
                          preferred_element_type=jnp.float32) + bk_ref[...])
    s0 = jnp.sum(t0 * q_ref[...]) * (1.0 / N)
    s1 = jnp.sum(t1 * q_ref[...]) * (1.0 / N)
    lane = lax.broadcasted_iota(jnp.int32, (1, 128), 1)
    vec = jnp.where(lane == 0, s0, 0.0) + jnp.where(lane == 1, s1, 0.0)

    @pl.when(i == 0)
    def _():
        out_ref[...] = vec

    @pl.when(i != 0)
    def _():
        out_ref[...] = out_ref[...] + vec


_score = pl.pallas_call(
    _score_body,
    grid=(N // BN,),
    in_specs=[pl.BlockSpec((BN, C), lambda i: (i, 0)),
              pl.BlockSpec((BN, C), lambda i: (i, 0)),
              pl.BlockSpec((C, C), lambda i: (0, 0)),
              pl.BlockSpec((1, C), lambda i: (0, 0)),
              pl.BlockSpec((1, C), lambda i: (0, 0))],
    out_specs=pl.BlockSpec((1, 128), lambda i: (0, 0)),
    out_shape=jax.ShapeDtypeStruct((1, 128), jnp.float32),
)


def _combine_body(sc_ref, o0_ref, o1_ref, out_ref):
    scv = sc_ref[...]
    lane = lax.broadcasted_iota(jnp.int32, (1, 128), 1)
    s0 = jnp.sum(jnp.where(lane == 0, scv, 0.0))
    s1 = jnp.sum(jnp.where(lane == 1, scv, 0.0))
    m = jnp.maximum(s0, s1)
    e0 = jnp.exp(s0 - m)
    e1 = jnp.exp(s1 - m)
    a0 = e0 / (e0 + e1)
    a1 = e1 / (e0 + e1)
    out_ref[...] = a0 * o0_ref[...] + a1 * o1_ref[...]


_combine = pl.pallas_call(
    _combine_body,
    grid=(N // BN,),
    in_specs=[pl.BlockSpec((1, 128), lambda i: (0, 0)),
              pl.BlockSpec((BN, C), lambda i: (i, 0)),
              pl.BlockSpec((BN, C), lambda i: (i, 0))],
    out_specs=pl.BlockSpec((BN, C), lambda i: (i, 0)),
    out_shape=jax.ShapeDtypeStruct((N, C), jnp.float32),
)


def _att_mat(att):
    m = (jnp.eye(H, dtype=jnp.float32)[:, None, :]
         * att[0][:, :, None]).reshape(C, H)
    return jnp.pad(m, ((0, 0), (0, AP - H)))


def _flat_pad(ad):
    return jnp.pad(ad, ((0, OUTP - N), (0, 0))).reshape(-1)


def _layer(x_a, x_p, edges, p):
    w_a = p["proj"]["author"]["W"]
    b_a = p["proj"]["author"]["b"][None, :]
    w_p = p["proj"]["paper"]["W"]
    b_p = p["proj"]["paper"]["b"][None, :]

    A_w_src = _att_mat(p["att"]["writes"]["src"])
    A_w_dst = _att_mat(p["att"]["writes"]["dst"])
    A_wb_src = _att_mat(p["att"]["written_by"]["src"])
    A_wb_dst = _att_mat(p["att"]["written_by"]["dst"])
    A_c_src = _att_mat(p["att"]["cites"]["src"])
    A_c_dst = _att_mat(p["att"]["cites"]["dst"])
    Z = jnp.zeros((C, AP), jnp.float32)

    hx_a, ad_wb, _ = _proj(x_a, w_a, b_a, A_w_src, Z, A_wb_dst, Z)
    hx_p, ad_w, ad_c = _proj(x_p, w_p, b_p, A_wb_src, A_c_src, A_w_dst, A_c_dst)

    o_writes = _edge_conv(edges["writes"], _edge_pass_s0, _flat_pad(ad_w), hx_a)
    o_writtenby = _edge_conv(edges["written_by"], _edge_pass_s0,
                             _flat_pad(ad_wb), hx_p)
    o_cites = _edge_conv(edges["cites"], _edge_pass_s1, _flat_pad(ad_c), hx_p)

    new_a = o_writtenby

    wk = p["k_lin"]["W"]
    bk = p["k_lin"]["b"][None, :]
    q = p["q"]
    scores = _score(o_writes, o_cites, wk, bk, q)
    new_p = _combine(scores, o_writes, o_cites)
    return new_a, new_p


def kernel(x_author, x_paper, edge_index_writes, edge_index_written_by,
           edge_index_cites, batch, params):
    edges = {
        "writes": _bucketize(edge_index_writes),
        "written_by": _bucketize(edge_index_written_by),
        "cites": _bucketize(edge_index_cites),
    }
    x_a, x_p = x_author, x_paper
    for p in params:
        x_a, x_p = _layer(x_a, x_p, edges, p)
    return (x_a[None], x_p[None])


# speedup vs baseline: 1.1165x; 1.1165x over previous
import functools

import jax
import jax.numpy as jnp
from jax import lax
from jax.experimental import pallas as pl
from jax.experimental.pallas import tpu as pltpu
from jax.experimental.pallas import tpu_sc as plsc

N = 10000
C = 256
H = 8
D = 32
E = 160000
AP = 16
LW = 128

NW = 32
ROWS = 320
OUTP = NW * ROWS
BLK = 96
PE = E + 384
VL = 16
XC = 384


def _make_edge_body(w_off):
    def body(pk_h, st_h, ct_h, adfl_h, hx_h, zero_h, out_h,
             acc, s_flat, adst_loc, idx_rc, idx_adj, idx_lkp, hbuf, mbuf):
        c = lax.axis_index("c")
        s = lax.axis_index("s")
        wid = c * 16 + s
        base = wid * ROWS

        pltpu.sync_copy(st_h.at[pl.ds(wid * 8, VL)], mbuf)
        st = pl.multiple_of(mbuf[...][0], 8)
        pltpu.sync_copy(ct_h.at[pl.ds(wid * 8, VL)], mbuf)
        cnt = mbuf[...][0]
        nb = (cnt + (BLK - 1)) // BLK

        pltpu.sync_copy(adfl_h.at[pl.ds(base * AP, ROWS * AP)], adst_loc)
        pltpu.sync_copy(zero_h.at[pl.ds(0, ROWS * C)], acc.at[pl.ds(0, ROWS * C)])
        pltpu.sync_copy(zero_h.at[pl.ds(0, ROWS * AP)],
                        s_flat.at[pl.ds(0, ROWS * AP)])

        def blk(b, carry):
            off = st + b * BLK
            pltpu.sync_copy(pk_h.at[pl.ds(off, BLK)], idx_rc)
            for i in range(BLK // VL):
                pv = idx_rc[pl.ds(i * VL, VL)]
                cv = pv >> 16
                rv = pv & 0xFFFF
                adj = cv - base
                ok = (adj >= 0) & (adj < ROWS)
                idx_adj[pl.ds(i * VL, VL)] = jnp.where(ok, adj, ROWS)
                idx_lkp[pl.ds(i * VL, VL)] = jnp.where(ok, adj, 0)
                idx_rc[pl.ds(i * VL, VL)] = rv
            pltpu.sync_copy(hx_h.at[idx_rc], hbuf)

            def grp(g, cg):
                iav = idx_adj[pl.ds(g * VL, VL)]
                ilv = idx_lkp[pl.ds(g * VL, VL)]
                for j in range(VL):
                    e = g * VL + j
                    av = (hbuf[e, pl.ds(w_off, VL)]
                          + adst_loc[pl.ds(ilv[j] * AP, VL)])
                    av = jnp.where(av >= 0.0, av, 0.2 * av)
                    wv = jnp.exp(av)
                    sofs = iav[j] * AP
                    s_flat[pl.ds(sofs, VL)] = s_flat[pl.ds(sofs, VL)] + wv
                    aofs = iav[j] * C
                    for hd in range(H):
                        w = wv[hd]
                        for q in range(2):
                            so = hd * D + q * VL
                            dsl = pl.ds(aofs + so, VL)
                            acc[dsl] = acc[dsl] + hbuf[e, pl.ds(so, VL)] * w
                return cg

            lax.fori_loop(0, BLK // VL, grp, 0)
            return carry

        lax.fori_loop(0, nb, blk, 0)

        def nrm(r, cn):
            sv = s_flat[pl.ds(r * VL, VL)]
            iv = 1.0 / (sv + 1e-16)
            for hd in range(H):
                w = iv[hd]
                for q in range(2):
                    so = r * C + hd * D + q * VL
                    v = acc[pl.ds(so, VL)] * w
                    acc[pl.ds(so, VL)] = jnp.maximum(v, 0.0)
            return cn

        lax.fori_loop(0, ROWS, nrm, 0)
        pltpu.sync_copy(acc.at[pl.ds(0, ROWS * C)],
                        out_h.at[pl.ds(base * C, ROWS * C)])

    return body


def _make_edge_pass(w_off):
    return functools.partial(
        pl.kernel,
        mesh=plsc.VectorSubcoreMesh(core_axis_name="c", subcore_axis_name="s"),
        out_type=jax.ShapeDtypeStruct((OUTP * C,), jnp.float32),
        scratch_types=[
            pltpu.VMEM(((ROWS + 1) * C,), jnp.float32),
            pltpu.VMEM(((ROWS + 1) * AP,), jnp.float32),
            pltpu.VMEM((ROWS * AP,), jnp.float32),
            pltpu.VMEM((BLK,), jnp.int32),
            pltpu.VMEM((BLK,), jnp.int32),
            pltpu.VMEM((BLK,), jnp.int32),
            pltpu.VMEM((BLK, XC), jnp.float32),
            pltpu.VMEM((VL,), jnp.int32),
        ],
    )(_make_edge_body(w_off))


_edge_pass_s0 = _make_edge_pass(C)
_edge_pass_s1 = _make_edge_pass(C + AP)


_ZEROS = None


def _edge_conv(bkt, edge_pass, adfl, hx, zeros):
    pk, st8, ct8 = bkt
    out = edge_pass(pk, st8, ct8, adfl, hx, zeros)
    return out.reshape(OUTP, C)[:N]


def _bucketize(ei):
    row, col = ei[0], ei[1]
    key = col // ROWS
    order = jnp.argsort(key)
    row_s = jnp.take(row, order)
    col_s = jnp.take(col, order)
    key_s = jnp.take(key, order)
    cnt = jnp.bincount(key, length=NW).astype(jnp.int32)
    pcnt = ((cnt + 7) // 8) * 8
    st = (jnp.cumsum(pcnt) - pcnt).astype(jnp.int32)
    cst = (jnp.cumsum(cnt) - cnt).astype(jnp.int32)
    rank = jnp.arange(E, dtype=jnp.int32) - jnp.take(cst, key_s)
    dest = jnp.take(st, key_s) + rank
    pk_s = (col_s << 16) | row_s
    pk = jnp.full((PE,), -65536, jnp.int32).at[dest].set(pk_s)
    st8 = jnp.pad(jnp.repeat(st, 8), (0, 16))
    ct8 = jnp.pad(jnp.repeat(cnt, 8), (0, 16))
    return pk, st8, ct8


BN = 400


def _proj_body(x_ref, w_ref, b_ref, as0_ref, as1_ref, ad0_ref, ad1_ref,
               hx_ref, od0_ref, od1_ref):
    h = jnp.dot(x_ref[...], w_ref[...],
                preferred_element_type=jnp.float32) + b_ref[...]
    a_s0 = jnp.dot(h, as0_ref[...], preferred_element_type=jnp.float32)
    a_s1 = jnp.dot(h, as1_ref[...], preferred_element_type=jnp.float32)
    zz = jnp.zeros((BN, XC - C - 2 * AP), jnp.float32)
    hx_ref[...] = jnp.concatenate([h, a_s0, a_s1, zz], axis=1)
    od0_ref[...] = jnp.dot(h, ad0_ref[...], preferred_element_type=jnp.float32)
    od1_ref[...] = jnp.dot(h, ad1_ref[...], preferred_element_type=jnp.float32)


_proj = pl.pallas_call(
    _proj_body,
    grid=(N // BN,),
    in_specs=[pl.BlockSpec((BN, C), lambda i: (i, 0)),
              pl.BlockSpec((C, C), lambda i: (0, 0)),
              pl.BlockSpec((1, C), lambda i: (0, 0))]
             + [pl.BlockSpec((C, AP), lambda i: (0, 0))] * 4,
    out_specs=[pl.BlockSpec((BN, XC), lambda i: (i, 0)),
               pl.BlockSpec((BN, AP), lambda i: (i, 0)),
               pl.BlockSpec((BN, AP), lambda i: (i, 0))],
    out_shape=[jax.ShapeDtypeStruct((N, XC), jnp.float32),
               jax.ShapeDtypeStruct((N, AP), jnp.float32),
               jax.ShapeDtypeStruct((N, AP), jnp.float32)],
)


def _score_body(o0_ref, o1_ref, wk_ref, bk_ref, q_ref, out_ref):
    i = pl.program_id(0)
    t0 = jnp.tanh(jnp.dot(o0_ref[...], wk_ref[...],
                          preferred_element_type=jnp.float32) + bk_ref[...])
    t1 = jnp.tanh(jnp.dot(o1_ref[...], wk_ref[...],
                          preferred_element_type=jnp.float32) + bk_ref[...])
    s0 = jnp.sum(t0 * q_ref[...]) * (1.0 / N)
    s1 = jnp.sum(t1 * q_ref[...]) * (1.0 / N)
    lane = lax.broadcasted_iota(jnp.int32, (1, 128), 1)
    vec = jnp.where(lane == 0, s0, 0.0) + jnp.where(lane == 1, s1, 0.0)

    @pl.when(i == 0)
    def _():
        out_ref[...] = vec

    @pl.when(i != 0)
    def _():
        out_ref[...] = out_ref[...] + vec


_score = pl.pallas_call(
    _score_body,
    grid=(N // BN,),
    in_specs=[pl.BlockSpec((BN, C), lambda i: (i, 0)),
              pl.BlockSpec((BN, C), lambda i: (i, 0)),
              pl.BlockSpec((C, C), lambda i: (0, 0)),
              pl.BlockSpec((1, C), lambda i: (0, 0)),
              pl.BlockSpec((1, C), lambda i: (0, 0))],
    out_specs=pl.BlockSpec((1, 128), lambda i: (0, 0)),
    out_shape=jax.ShapeDtypeStruct((1, 128), jnp.float32),
)


def _combine_body(sc_ref, o0_ref, o1_ref, out_ref):
    scv = sc_ref[...]
    lane = lax.broadcasted_iota(jnp.int32, (1, 128), 1)
    s0 = jnp.sum(jnp.where(lane == 0, scv, 0.0))
    s1 = jnp.sum(jnp.where(lane == 1, scv, 0.0))
    m = jnp.maximum(s0, s1)
    e0 = jnp.exp(s0 - m)
    e1 = jnp.exp(s1 - m)
    a0 = e0 / (e0 + e1)
    a1 = e1 / (e0 + e1)
    out_ref[...] = a0 * o0_ref[...] + a1 * o1_ref[...]


_combine = pl.pallas_call(
    _combine_body,
    grid=(N // BN,),
    in_specs=[pl.BlockSpec((1, 128), lambda i: (0, 0)),
              pl.BlockSpec((BN, C), lambda i: (i, 0)),
              pl.BlockSpec((BN, C), lambda i: (i, 0))],
    out_specs=pl.BlockSpec((BN, C), lambda i: (i, 0)),
    out_shape=jax.ShapeDtypeStruct((N, C), jnp.float32),
)


def _att_mat(att):
    m = (jnp.eye(H, dtype=jnp.float32)[:, None, :]
         * att[0][:, :, None]).reshape(C, H)
    return jnp.pad(m, ((0, 0), (0, AP - H)))


def _flat_pad(ad):
    return jnp.pad(ad, ((0, OUTP - N), (0, 0))).reshape(-1)


def _layer(x_a, x_p, edges, p):
    w_a = p["proj"]["author"]["W"]
    b_a = p["proj"]["author"]["b"][None, :]
    w_p = p["proj"]["paper"]["W"]
    b_p = p["proj"]["paper"]["b"][None, :]

    A_w_src = _att_mat(p["att"]["writes"]["src"])
    A_w_dst = _att_mat(p["att"]["writes"]["dst"])
    A_wb_src = _att_mat(p["att"]["written_by"]["src"])
    A_wb_dst = _att_mat(p["att"]["written_by"]["dst"])
    A_c_src = _att_mat(p["att"]["cites"]["src"])
    A_c_dst = _att_mat(p["att"]["cites"]["dst"])
    Z = jnp.zeros((C, AP), jnp.float32)

    hx_a, ad_wb, _ = _proj(x_a, w_a, b_a, A_w_src, Z, A_wb_dst, Z)
    hx_p, ad_w, ad_c = _proj(x_p, w_p, b_p, A_wb_src, A_c_src, A_w_dst, A_c_dst)

    zeros = jnp.zeros((ROWS * C,), jnp.float32)
    o_writes = _edge_conv(edges["writes"], _edge_pass_s0, _flat_pad(ad_w),
                          hx_a, zeros)
    o_writtenby = _edge_conv(edges["written_by"], _edge_pass_s0,
                             _flat_pad(ad_wb), hx_p, zeros)
    o_cites = _edge_conv(edges["cites"], _edge_pass_s1, _flat_pad(ad_c),
                         hx_p, zeros)

    new_a = o_writtenby

    wk = p["k_lin"]["W"]
    bk = p["k_lin"]["b"][None, :]
    q = p["q"]
    scores = _score(o_writes, o_cites, wk, bk, q)
    new_p = _combine(scores, o_writes, o_cites)
    return new_a, new_p


def kernel(x_author, x_paper, edge_index_writes, edge_index_written_by,
           edge_index_cites, batch, params):
    edges = {
        "writes": _bucketize(edge_index_writes),
        "written_by": _bucketize(edge_index_written_by),
        "cites": _bucketize(edge_index_cites),
    }
    x_a, x_p = x_author, x_paper
    for p in params:
        x_a, x_p = _layer(x_a, x_p, edges, p)
    return (x_a[None], x_p[None])
